# baseline (speedup 1.0000x reference)
import jax
import jax.numpy as jnp
from jax.experimental import pallas as pl
from jax.experimental.pallas import tpu as pltpu

N = 8192
K = 16
RB = 256
IMAX = 2**31 - 1


def _knn_body(rows_ref, cols_ref, pc_ref, out_ref, keys_ref):
    xi = rows_ref[:, 0:1]
    yi = rows_ref[:, 1:2]
    zi = rows_ref[:, 2:3]
    xj = cols_ref[0:1, :]
    yj = cols_ref[1:2, :]
    zj = cols_ref[2:3, :]
    dx = xi - xj
    dy = yi - yj
    dz = zi - zj
    d2 = dx * dx + dy * dy + dz * dz
    j = jax.lax.broadcasted_iota(jnp.int32, (RB, N), 1)
    keys_ref[...] = d2 * N + j
    pcb = pc_ref[...]
    sel = []
    for t in range(K):
        keys = keys_ref[...]
        m = jnp.min(keys, axis=1, keepdims=True)
        eq = keys == m
        psel = jnp.min(jnp.where(eq, pcb, IMAX), axis=1, keepdims=True)
        sel.append(psel)
        if t + 1 < K:
            keys_ref[...] = jnp.where(eq, IMAX, keys)
    out_ref[...] = jnp.concatenate(sel, axis=1)


MB = 1024


def _mlp1_body(pc_ref, w1_ref, b1_ref, h_ref, sums_ref):
    pc = pc_ref[...]
    x = pc >> 14
    y = (pc >> 7) & 127
    z = pc & 127
    relx = (x[:, 1:] - x[:, 0:1]).astype(jnp.float32)
    rely = (y[:, 1:] - y[:, 0:1]).astype(jnp.float32)
    relz = (z[:, 1:] - z[:, 0:1]).astype(jnp.float32)
    rel = jnp.concatenate([relx, rely, relz], axis=1)
    h = jax.lax.dot_general(
        rel, w1_ref[...], (((1,), (0,)), ((), ())),
        precision=jax.lax.Precision.HIGHEST,
        preferred_element_type=jnp.float32) + b1_ref[...]
    h_ref[...] = h
    ps = jnp.sum(h, axis=0, keepdims=True)
    ps2 = jnp.sum(h * h, axis=0, keepdims=True)
    part = jnp.concatenate([ps, ps2], axis=0)

    @pl.when(pl.program_id(0) == 0)
    def _():
        sums_ref[...] = part

    @pl.when(pl.program_id(0) != 0)
    def _():
        sums_ref[...] += part


def _mlp2_body(sums_ref, h_ref, gamma_ref, beta_ref, w2_ref, b2_ref, out_ref):
    inv_n = 1.0 / N
    mean = sums_ref[0:1, :] * inv_n
    var = sums_ref[1:2, :] * inv_n - mean * mean
    scale = gamma_ref[...] / jnp.sqrt(var + 1e-5)
    shift = beta_ref[...] - mean * scale
    hn = jnp.maximum(h_ref[...] * scale + shift, 0.0)
    out_ref[...] = jax.lax.dot_general(
        hn, w2_ref[...], (((1,), (0,)), ((), ())),
        precision=jax.lax.Precision.HIGHEST,
        preferred_element_type=jnp.float32) + b2_ref[...]


def _run(coords, colsT, pc, W1p, b1, gamma, beta, W2, b2, interpret=False):
    npf = W1p.shape[1]
    selpc = pl.pallas_call(
        _knn_body,
        grid=(N // RB,),
        in_specs=[
            pl.BlockSpec((RB, 3), lambda i: (i, 0)),
            pl.BlockSpec((3, N), lambda i: (0, 0)),
            pl.BlockSpec((1, N), lambda i: (0, 0)),
        ],
        out_specs=pl.BlockSpec((RB, K), lambda i: (i, 0)),
        out_shape=jax.ShapeDtypeStruct((N, K), jnp.int32),
        scratch_shapes=[pltpu.VMEM((RB, N), jnp.int32)],
        compiler_params=pltpu.CompilerParams(
            dimension_semantics=("parallel",)),
        interpret=interpret,
    )(coords, colsT, pc)

    nblk = N // MB
    h, sums = pl.pallas_call(
        _mlp1_body,
        grid=(nblk,),
        in_specs=[
            pl.BlockSpec((MB, K), lambda i: (i, 0)),
            pl.BlockSpec((3 * (K - 1), npf), lambda i: (0, 0)),
            pl.BlockSpec((1, npf), lambda i: (0, 0)),
        ],
        out_specs=[
            pl.BlockSpec((MB, npf), lambda i: (i, 0)),
            pl.BlockSpec((2, npf), lambda i: (0, 0)),
        ],
        out_shape=[
            jax.ShapeDtypeStruct((N, npf), jnp.float32),
            jax.ShapeDtypeStruct((2, npf), jnp.float32),
        ],
        interpret=interpret,
    )(selpc, W1p, b1)

    out = pl.pallas_call(
        _mlp2_body,
        grid=(nblk,),
        in_specs=[
            pl.BlockSpec((2, npf), lambda i: (0, 0)),
            pl.BlockSpec((MB, npf), lambda i: (i, 0)),
            pl.BlockSpec((1, npf), lambda i: (0, 0)),
            pl.BlockSpec((1, npf), lambda i: (0, 0)),
            pl.BlockSpec((npf, npf), lambda i: (0, 0)),
            pl.BlockSpec((1, npf), lambda i: (0, 0)),
        ],
        out_specs=pl.BlockSpec((MB, npf), lambda i: (i, 0)),
        out_shape=jax.ShapeDtypeStruct((N, npf), jnp.float32),
        interpret=interpret,
    )(sums, h, gamma, beta, W2, b2)
    return out


def kernel(indices, W1, b1, gamma, beta, W2, b2):
    coords = indices[:, 1:].astype(jnp.int32)
    colsT = coords.T
    pc = ((coords[:, 0] * 128 + coords[:, 1]) * 128
          + coords[:, 2]).reshape(1, N)
    npf = W1.shape[1]
    W1p = W1.reshape(K - 1, 3, npf).transpose(1, 0, 2).reshape(3 * (K - 1), npf)
    return _run(coords, colsT, pc, W1p,
                b1.reshape(1, npf), gamma.reshape(1, npf),
                beta.reshape(1, npf), W2, b2.reshape(1, npf))


# speedup vs baseline: 2.0545x; 2.0545x over previous
import dataclasses
import functools

import jax
import jax.numpy as jnp
from jax.experimental import pallas as pl
from jax.experimental.pallas import tpu as pltpu
from jax.experimental.pallas import tpu_sc as plsc

N = 8192
K = 16
RB = 256
IMAX = 2**31 - 1


def _knn_body(rows_ref, cols_ref, out_ref, keys_ref):
    xi = rows_ref[:, 0:1]
    yi = rows_ref[:, 1:2]
    zi = rows_ref[:, 2:3]
    xj = cols_ref[0:1, :]
    yj = cols_ref[1:2, :]
    zj = cols_ref[2:3, :]
    dx = xi - xj
    dy = yi - yj
    dz = zi - zj
    d2 = dx * dx + dy * dy + dz * dz
    j = jax.lax.broadcasted_iota(jnp.int32, (RB, N), 1)
    keys_ref[...] = d2 * N + j
    sel = []
    for t in range(K):
        keys = keys_ref[...]
        m = jnp.min(keys, axis=1, keepdims=True)
        sel.append(m & (N - 1))
        if t + 1 < K:
            keys_ref[...] = jnp.where(keys == m, IMAX, keys)
    out_ref[...] = jnp.concatenate(sel, axis=1)


def _sc_gather(pc, idx_flat, interpret=False):
    info = plsc.get_sparse_core_info()
    nw = info.num_cores * info.num_subcores
    b_per_w = idx_flat.shape[0] // nw
    mesh = plsc.VectorSubcoreMesh(core_axis_name="c", subcore_axis_name="s")
    cp = pltpu.CompilerParams()
    if "needs_layout_passes" in pltpu.CompilerParams.__dataclass_fields__:
        cp = dataclasses.replace(cp, needs_layout_passes=False)

    @functools.partial(
        pl.kernel, mesh=mesh,
        out_type=jax.ShapeDtypeStruct((idx_flat.shape[0],), jnp.int32),
        scratch_types=[
            pltpu.VMEM((N,), jnp.int32),
            pltpu.VMEM((b_per_w,), jnp.int32),
            pltpu.VMEM((b_per_w,), jnp.int32),
        ],
        compiler_params=cp,
        interpret=interpret,
    )
    def k(pc_hbm, idx_hbm, out_hbm, pc_v, idx_v, out_v):
        wid = jax.lax.axis_index("s") * info.num_cores + jax.lax.axis_index("c")
        base = wid * b_per_w
        pltpu.sync_copy(pc_hbm, pc_v)
        pltpu.sync_copy(idx_hbm.at[pl.ds(base, b_per_w)], idx_v)

        @pl.loop(0, b_per_w, step=16)
        def _(i):
            j16 = idx_v[pl.ds(i, 16)]
            out_v[pl.ds(i, 16)] = plsc.load_gather(pc_v, [j16])

        pltpu.sync_copy(out_v, out_hbm.at[pl.ds(base, b_per_w)])

    return k(pc, idx_flat)


MB = 1024


def _mlp1_body(pc_ref, w1_ref, b1_ref, h_ref, sums_ref):
    pc = pc_ref[...]
    x = pc >> 14
    y = (pc >> 7) & 127
    z = pc & 127
    relx = (x[:, 1:] - x[:, 0:1]).astype(jnp.float32)
    rely = (y[:, 1:] - y[:, 0:1]).astype(jnp.float32)
    relz = (z[:, 1:] - z[:, 0:1]).astype(jnp.float32)
    rel = jnp.concatenate([relx, rely, relz], axis=1)
    h = jax.lax.dot_general(
        rel, w1_ref[...], (((1,), (0,)), ((), ())),
        precision=jax.lax.Precision.HIGHEST,
        preferred_element_type=jnp.float32) + b1_ref[...]
    h_ref[...] = h
    ps = jnp.sum(h, axis=0, keepdims=True)
    ps2 = jnp.sum(h * h, axis=0, keepdims=True)
    part = jnp.concatenate([ps, ps2], axis=0)

    @pl.when(pl.program_id(0) == 0)
    def _():
        sums_ref[...] = part

    @pl.when(pl.program_id(0) != 0)
    def _():
        sums_ref[...] += part


def _mlp2_body(sums_ref, h_ref, gamma_ref, beta_ref, w2_ref, b2_ref, out_ref):
    inv_n = 1.0 / N
    mean = sums_ref[0:1, :] * inv_n
    var = sums_ref[1:2, :] * inv_n - mean * mean
    scale = gamma_ref[...] / jnp.sqrt(var + 1e-5)
    shift = beta_ref[...] - mean * scale
    hn = jnp.maximum(h_ref[...] * scale + shift, 0.0)
    out_ref[...] = jax.lax.dot_general(
        hn, w2_ref[...], (((1,), (0,)), ((), ())),
        precision=jax.lax.Precision.HIGHEST,
        preferred_element_type=jnp.float32) + b2_ref[...]


def _run(coords, colsT, pc, W1p, b1, gamma, beta, W2, b2, interpret=False):
    npf = W1p.shape[1]
    idx = pl.pallas_call(
        _knn_body,
        grid=(N // RB,),
        in_specs=[
            pl.BlockSpec((RB, 3), lambda i: (i, 0)),
            pl.BlockSpec((3, N), lambda i: (0, 0)),
        ],
        out_specs=pl.BlockSpec((RB, K), lambda i: (i, 0)),
        out_shape=jax.ShapeDtypeStruct((N, K), jnp.int32),
        scratch_shapes=[pltpu.VMEM((RB, N), jnp.int32)],
        compiler_params=pltpu.CompilerParams(
            dimension_semantics=("parallel",)),
        interpret=interpret,
    )(coords, colsT)

    selpc = _sc_gather(pc.reshape(N), idx.reshape(N * K),
                       interpret=interpret).reshape(N, K)

    nblk = N // MB
    h, sums = pl.pallas_call(
        _mlp1_body,
        grid=(nblk,),
        in_specs=[
            pl.BlockSpec((MB, K), lambda i: (i, 0)),
            pl.BlockSpec((3 * (K - 1), npf), lambda i: (0, 0)),
            pl.BlockSpec((1, npf), lambda i: (0, 0)),
        ],
        out_specs=[
            pl.BlockSpec((MB, npf), lambda i: (i, 0)),
            pl.BlockSpec((2, npf), lambda i: (0, 0)),
        ],
        out_shape=[
            jax.ShapeDtypeStruct((N, npf), jnp.float32),
            jax.ShapeDtypeStruct((2, npf), jnp.float32),
        ],
        interpret=interpret,
    )(selpc, W1p, b1)

    out = pl.pallas_call(
        _mlp2_body,
        grid=(nblk,),
        in_specs=[
            pl.BlockSpec((2, npf), lambda i: (0, 0)),
            pl.BlockSpec((MB, npf), lambda i: (i, 0)),
            pl.BlockSpec((1, npf), lambda i: (0, 0)),
            pl.BlockSpec((1, npf), lambda i: (0, 0)),
            pl.BlockSpec((npf, npf), lambda i: (0, 0)),
            pl.BlockSpec((1, npf), lambda i: (0, 0)),
        ],
        out_specs=pl.BlockSpec((MB, npf), lambda i: (i, 0)),
        out_shape=jax.ShapeDtypeStruct((N, npf), jnp.float32),
        interpret=interpret,
    )(sums, h, gamma, beta, W2, b2)
    return out


def kernel(indices, W1, b1, gamma, beta, W2, b2):
    coords = indices[:, 1:].astype(jnp.int32)
    colsT = coords.T
    pc = ((coords[:, 0] * 128 + coords[:, 1]) * 128
          + coords[:, 2]).reshape(1, N)
    npf = W1.shape[1]
    W1p = W1.reshape(K - 1, 3, npf).transpose(1, 0, 2).reshape(3 * (K - 1), npf)
    return _run(coords, colsT, pc, W1p,
                b1.reshape(1, npf), gamma.reshape(1, npf),
                beta.reshape(1, npf), W2, b2.reshape(1, npf))
